# baseline (device time: 80126 ns/iter reference)
import jax
import jax.numpy as jnp
from jax import lax
from jax.experimental import pallas as pl
from jax.experimental.pallas import tpu as pltpu

N_DEV = 8
SQ = 256
SKV = 4096
HQ = 8
DH = 128
D = HQ * DH
SCALE = 0.08838834764831843
COLS = D + DH
OWN = SQ // N_DEV
QB = 64
NG = SKV // (4 * QB)


def kernel(x, Wq, K_ext, V_ext, Wo):
    def body(x_ref, wq_ref, k_ref, v_ref, wo_ref, out_ref,
             part_ref, s0_ref, s1_ref, s2_ref, r0_ref, r1_ref, r2_ref,
             ag_ref, send_sems, recv_sems, ag_send_sems, ag_recv_sems):
        my = lax.axis_index("i")

        q = jnp.dot(x_ref[0], wq_ref[...],
                    preferred_element_type=jnp.float32) * SCALE

        for c in range(4):
            kgc = jnp.concatenate(
                [k_ref[0, (4 * g + c) * QB:(4 * g + c + 1) * QB, :]
                 for g in range(NG)], axis=0)
            vgc = jnp.concatenate(
                [v_ref[0, (4 * g + c) * QB:(4 * g + c + 1) * QB, :]
                 for g in range(NG)], axis=0)
            for h in range(HQ):
                qc = q[c * QB:(c + 1) * QB, h * DH:(h + 1) * DH]
                s = lax.dot_general(
                    qc, kgc[:, h * DH:(h + 1) * DH],
                    (((1,), (1,)), ((), ())),
                    preferred_element_type=jnp.float32)
                w = jnp.exp(s)
                part_ref[c * QB:(c + 1) * QB, D + h:D + h + 1] = jnp.sum(
                    w, axis=1, keepdims=True)
                part_ref[c * QB:(c + 1) * QB, h * DH:(h + 1) * DH] = jnp.dot(
                    w, vgc[:, h * DH:(h + 1) * DH],
                    preferred_element_type=jnp.float32)
        part_ref[:, D + HQ:COLS] = jnp.zeros((SQ, COLS - D - HQ),
                                             jnp.float32)

        pending = []

        barrier_sem = pltpu.get_barrier_semaphore()
        for m in (1, 2, 4):
            pl.semaphore_signal(barrier_sem, inc=1,
                                device_id=(lax.bitwise_xor(my, m),),
                                device_id_type=pl.DeviceIdType.MESH)
        pl.semaphore_wait(barrier_sem, 3)

        base = my * 0
        stages = (s0_ref, s1_ref, s2_ref)
        recvs = (r0_ref, r1_ref, r2_ref)
        for j, m in enumerate((4, 2, 1)):
            sz = SQ >> (j + 1)
            bit = lax.bitwise_and(lax.shift_right_logical(my, 2 - j), 1)
            keep_off = pl.multiple_of(base + bit * sz, 32)
            send_off = pl.multiple_of(base + (1 - bit) * sz, 32)
            stages[j][...] = part_ref[pl.ds(send_off, sz), :].astype(
                jnp.bfloat16)
            rdma = pltpu.make_async_remote_copy(
                src_ref=stages[j],
                dst_ref=recvs[j],
                send_sem=send_sems.at[j],
                recv_sem=recv_sems.at[j],
                device_id=(lax.bitwise_xor(my, m),),
                device_id_type=pl.DeviceIdType.MESH,
            )
            rdma.start()
            rdma.wait_recv()
            pending.append(rdma)
            part_ref[pl.ds(keep_off, sz), :] = (
                part_ref[pl.ds(keep_off, sz), :]
                + recvs[j][...].astype(jnp.float32))
            base = keep_off
        base = pl.multiple_of(base, 32)

        mine = part_ref[pl.ds(base, OWN), :]
        ctx_cols = []
        for h in range(HQ):
            ctx_cols.append(mine[:, h * DH:(h + 1) * DH]
                            / mine[:, D + h:D + h + 1])
        ctx = jnp.concatenate(ctx_cols, axis=1)
        own = jnp.dot(ctx, wo_ref[...],
                      preferred_element_type=jnp.float32)
        ag_ref[pl.ds(base, OWN), :] = own.astype(jnp.bfloat16)

        own_off = base
        for j, m in enumerate((1, 2, 4)):
            sz = OWN << j
            own_off = pl.multiple_of(own_off, 32)
            rdma = pltpu.make_async_remote_copy(
                src_ref=ag_ref.at[pl.ds(own_off, sz), :],
                dst_ref=ag_ref.at[pl.ds(own_off, sz), :],
                send_sem=ag_send_sems.at[j],
                recv_sem=ag_recv_sems.at[j],
                device_id=(lax.bitwise_xor(my, m),),
                device_id_type=pl.DeviceIdType.MESH,
            )
            rdma.start()
            rdma.wait_recv()
            pending.append(rdma)
            own_off = own_off & ~sz

        out_ref[0] = ag_ref[...].astype(jnp.float32)
        for rdma in pending:
            rdma.wait_send()

    return pl.pallas_call(
        body,
        out_shape=jax.ShapeDtypeStruct((1, SQ, D), jnp.float32),
        in_specs=[pl.BlockSpec(memory_space=pltpu.VMEM)] * 5,
        out_specs=pl.BlockSpec(memory_space=pltpu.VMEM),
        scratch_shapes=[
            pltpu.VMEM((SQ, COLS), jnp.float32),
            pltpu.VMEM((SQ // 2, COLS), jnp.bfloat16),
            pltpu.VMEM((SQ // 4, COLS), jnp.bfloat16),
            pltpu.VMEM((SQ // 8, COLS), jnp.bfloat16),
            pltpu.VMEM((SQ // 2, COLS), jnp.bfloat16),
            pltpu.VMEM((SQ // 4, COLS), jnp.bfloat16),
            pltpu.VMEM((SQ // 8, COLS), jnp.bfloat16),
            pltpu.VMEM((SQ, D), jnp.bfloat16),
            pltpu.SemaphoreType.DMA((3,)),
            pltpu.SemaphoreType.DMA((3,)),
            pltpu.SemaphoreType.DMA((3,)),
            pltpu.SemaphoreType.DMA((3,)),
        ],
        compiler_params=pltpu.CompilerParams(
            collective_id=0, vmem_limit_bytes=100 * 1024 * 1024),
    )(x, Wq, K_ext.reshape(1, SKV, D), V_ext.reshape(1, SKV, D), Wo)


# device time: 48696 ns/iter; 1.6454x vs baseline; 1.6454x over previous
import jax
import jax.numpy as jnp
from jax import lax
from jax.experimental import pallas as pl
from jax.experimental.pallas import tpu as pltpu

N_DEV = 8
SQ = 256
SKV = 4096
HQ = 8
DH = 128
D = HQ * DH
SCALE = 0.08838834764831843
COLS = D + DH
OWN = SQ // N_DEV
QB = 64
NG = SKV // (4 * QB)


def kernel(x, Wq, K_ext, V_ext, Wo):
    def body(x_ref, wq_ref, k_ref, v_ref, wo_ref, out_ref,
             part_ref, s0_ref, s1_ref, s2_ref, r0_ref, r1_ref, r2_ref,
             ag_ref, kflat_ref, vflat_ref,
             send_sems, recv_sems, ag_send_sems, ag_recv_sems, ldma_sems):
        my = lax.axis_index("i")

        ldmas = []
        for h in range(HQ):
            for t, (src, dst) in enumerate(((k_ref, kflat_ref),
                                            (v_ref, vflat_ref))):
                d = pltpu.make_async_copy(
                    src.at[0, :, h, :],
                    dst.at[:, h * DH:(h + 1) * DH],
                    ldma_sems.at[2 * h + t])
                d.start()
                ldmas.append(d)

        q = jnp.dot(x_ref[0], wq_ref[...],
                    preferred_element_type=jnp.float32) * SCALE

        for d in ldmas:
            d.wait()

        for c in range(4):
            kgc = jnp.concatenate(
                [kflat_ref[(4 * g + c) * QB:(4 * g + c + 1) * QB, :]
                 for g in range(NG)], axis=0)
            vgc = jnp.concatenate(
                [vflat_ref[(4 * g + c) * QB:(4 * g + c + 1) * QB, :]
                 for g in range(NG)], axis=0)
            for h in range(HQ):
                qc = q[c * QB:(c + 1) * QB, h * DH:(h + 1) * DH]
                s = lax.dot_general(
                    qc, kgc[:, h * DH:(h + 1) * DH],
                    (((1,), (1,)), ((), ())),
                    preferred_element_type=jnp.float32)
                w = jnp.exp(s)
                part_ref[c * QB:(c + 1) * QB, D + h:D + h + 1] = jnp.sum(
                    w, axis=1, keepdims=True)
                part_ref[c * QB:(c + 1) * QB, h * DH:(h + 1) * DH] = jnp.dot(
                    w, vgc[:, h * DH:(h + 1) * DH],
                    preferred_element_type=jnp.float32)
        part_ref[:, D + HQ:COLS] = jnp.zeros((SQ, COLS - D - HQ),
                                             jnp.float32)

        pending = []

        barrier_sem = pltpu.get_barrier_semaphore()
        for m in (1, 2, 4):
            pl.semaphore_signal(barrier_sem, inc=1,
                                device_id=(lax.bitwise_xor(my, m),),
                                device_id_type=pl.DeviceIdType.MESH)
        pl.semaphore_wait(barrier_sem, 3)

        base = my * 0
        stages = (s0_ref, s1_ref, s2_ref)
        recvs = (r0_ref, r1_ref, r2_ref)
        for j, m in enumerate((4, 2, 1)):
            sz = SQ >> (j + 1)
            bit = lax.bitwise_and(lax.shift_right_logical(my, 2 - j), 1)
            keep_off = pl.multiple_of(base + bit * sz, 32)
            send_off = pl.multiple_of(base + (1 - bit) * sz, 32)
            stages[j][...] = part_ref[pl.ds(send_off, sz), :].astype(
                jnp.bfloat16)
            rdma = pltpu.make_async_remote_copy(
                src_ref=stages[j],
                dst_ref=recvs[j],
                send_sem=send_sems.at[j],
                recv_sem=recv_sems.at[j],
                device_id=(lax.bitwise_xor(my, m),),
                device_id_type=pl.DeviceIdType.MESH,
            )
            rdma.start()
            rdma.wait_recv()
            pending.append(rdma)
            part_ref[pl.ds(keep_off, sz), :] = (
                part_ref[pl.ds(keep_off, sz), :]
                + recvs[j][...].astype(jnp.float32))
            base = keep_off
        base = pl.multiple_of(base, 32)

        mine = part_ref[pl.ds(base, OWN), :]
        ctx_cols = []
        for h in range(HQ):
            ctx_cols.append(mine[:, h * DH:(h + 1) * DH]
                            / mine[:, D + h:D + h + 1])
        ctx = jnp.concatenate(ctx_cols, axis=1)
        own = jnp.dot(ctx, wo_ref[...],
                      preferred_element_type=jnp.float32)
        ag_ref[pl.ds(base, OWN), :] = own.astype(jnp.bfloat16)

        own_off = base
        for j, m in enumerate((1, 2, 4)):
            sz = OWN << j
            own_off = pl.multiple_of(own_off, 32)
            rdma = pltpu.make_async_remote_copy(
                src_ref=ag_ref.at[pl.ds(own_off, sz), :],
                dst_ref=ag_ref.at[pl.ds(own_off, sz), :],
                send_sem=ag_send_sems.at[j],
                recv_sem=ag_recv_sems.at[j],
                device_id=(lax.bitwise_xor(my, m),),
                device_id_type=pl.DeviceIdType.MESH,
            )
            rdma.start()
            rdma.wait_recv()
            pending.append(rdma)
            own_off = own_off & ~sz

        out_ref[0] = ag_ref[...].astype(jnp.float32)
        for rdma in pending:
            rdma.wait_send()

    return pl.pallas_call(
        body,
        out_shape=jax.ShapeDtypeStruct((1, SQ, D), jnp.float32),
        in_specs=[
            pl.BlockSpec(memory_space=pltpu.VMEM),
            pl.BlockSpec(memory_space=pltpu.VMEM),
            pl.BlockSpec(memory_space=pl.ANY),
            pl.BlockSpec(memory_space=pl.ANY),
            pl.BlockSpec(memory_space=pltpu.VMEM),
        ],
        out_specs=pl.BlockSpec(memory_space=pltpu.VMEM),
        scratch_shapes=[
            pltpu.VMEM((SQ, COLS), jnp.float32),
            pltpu.VMEM((SQ // 2, COLS), jnp.bfloat16),
            pltpu.VMEM((SQ // 4, COLS), jnp.bfloat16),
            pltpu.VMEM((SQ // 8, COLS), jnp.bfloat16),
            pltpu.VMEM((SQ // 2, COLS), jnp.bfloat16),
            pltpu.VMEM((SQ // 4, COLS), jnp.bfloat16),
            pltpu.VMEM((SQ // 8, COLS), jnp.bfloat16),
            pltpu.VMEM((SQ, D), jnp.bfloat16),
            pltpu.VMEM((SKV, D), jnp.float32),
            pltpu.VMEM((SKV, D), jnp.float32),
            pltpu.SemaphoreType.DMA((3,)),
            pltpu.SemaphoreType.DMA((3,)),
            pltpu.SemaphoreType.DMA((3,)),
            pltpu.SemaphoreType.DMA((3,)),
            pltpu.SemaphoreType.DMA((2 * HQ,)),
        ],
        compiler_params=pltpu.CompilerParams(
            collective_id=0, vmem_limit_bytes=100 * 1024 * 1024),
    )(x, Wq, K_ext, V_ext, Wo)


# device time: 36663 ns/iter; 2.1855x vs baseline; 1.3282x over previous
import jax
import jax.numpy as jnp
from jax import lax
from jax.experimental import pallas as pl
from jax.experimental.pallas import tpu as pltpu

N_DEV = 8
SQ = 256
SKV = 4096
HQ = 8
DH = 128
D = HQ * DH
SCALE = 0.08838834764831843
COLS = D + DH
OWN = SQ // N_DEV
QB = 64
NG = SKV // (4 * QB)


def kernel(x, Wq, K_ext, V_ext, Wo):
    def body(x_ref, wq_ref, k_ref, v_ref, wo_ref, out_ref,
             part_ref, rs_stage_ref, rs_recv_ref, ag_stage_ref, ag_ref,
             kflat_ref, vflat_ref,
             rs_send_sems, rs_recv_sem, ag_send_sems, ag_recv_sem,
             ldma_sems):
        my = lax.axis_index("i")

        ldmas = []
        for h in range(HQ):
            for t, (src, dst) in enumerate(((k_ref, kflat_ref),
                                            (v_ref, vflat_ref))):
                dma = pltpu.make_async_copy(
                    src.at[0, :, h, :],
                    dst.at[:, h * DH:(h + 1) * DH],
                    ldma_sems.at[2 * h + t])
                dma.start()
                ldmas.append(dma)

        barrier_sem = pltpu.get_barrier_semaphore()
        for r in range(1, N_DEV):
            pl.semaphore_signal(
                barrier_sem, inc=1,
                device_id=(lax.rem(my + r, N_DEV),),
                device_id_type=pl.DeviceIdType.MESH)
        pl.semaphore_wait(barrier_sem, N_DEV - 1)

        q = jnp.dot(x_ref[0], wq_ref[...],
                    preferred_element_type=jnp.float32) * SCALE

        for dma in ldmas:
            dma.wait()

        for c in range(4):
            kgc = jnp.concatenate(
                [kflat_ref[(4 * g + c) * QB:(4 * g + c + 1) * QB, :]
                 for g in range(NG)], axis=0)
            vgc = jnp.concatenate(
                [vflat_ref[(4 * g + c) * QB:(4 * g + c + 1) * QB, :]
                 for g in range(NG)], axis=0)
            for h in range(HQ):
                qc = q[c * QB:(c + 1) * QB, h * DH:(h + 1) * DH]
                s = lax.dot_general(
                    qc, kgc[:, h * DH:(h + 1) * DH],
                    (((1,), (1,)), ((), ())),
                    preferred_element_type=jnp.float32)
                w = jnp.exp(s)
                part_ref[c * QB:(c + 1) * QB, D + h:D + h + 1] = jnp.sum(
                    w, axis=1, keepdims=True)
                part_ref[c * QB:(c + 1) * QB, h * DH:(h + 1) * DH] = jnp.dot(
                    w, vgc[:, h * DH:(h + 1) * DH],
                    preferred_element_type=jnp.float32)
            part_ref[c * QB:(c + 1) * QB, D + HQ:COLS] = jnp.zeros(
                (QB, COLS - D - HQ), jnp.float32)
            for d in (2 * c, 2 * c + 1):
                rs_stage_ref[d] = part_ref[
                    d * OWN:(d + 1) * OWN, :].astype(jnp.bfloat16)

                @pl.when(my != d)
                def _(d=d):
                    slot = lax.rem(jnp.int32(d + N_DEV - 1) - my, N_DEV)
                    pltpu.make_async_remote_copy(
                        src_ref=rs_stage_ref.at[d],
                        dst_ref=rs_recv_ref.at[slot],
                        send_sem=rs_send_sems.at[d],
                        recv_sem=rs_recv_sem,
                        device_id=(d,),
                        device_id_type=pl.DeviceIdType.MESH,
                    ).start()

        base = pl.multiple_of(my * OWN, 32)
        for t in range(N_DEV - 1):
            pltpu.make_async_copy(
                rs_recv_ref.at[t], rs_recv_ref.at[t], rs_recv_sem).wait()
        mine = part_ref[pl.ds(base, OWN), :]
        for t in range(N_DEV - 1):
            mine = mine + rs_recv_ref[t].astype(jnp.float32)

        ctx_cols = []
        for h in range(HQ):
            ctx_cols.append(mine[:, h * DH:(h + 1) * DH]
                            / mine[:, D + h:D + h + 1])
        ctx = jnp.concatenate(ctx_cols, axis=1)
        own = jnp.dot(ctx, wo_ref[...],
                      preferred_element_type=jnp.float32)
        ag_stage_ref[...] = own.astype(jnp.bfloat16)
        ag_ref[pl.ds(base, OWN), :] = ag_stage_ref[...]

        for r in range(1, N_DEV):
            pltpu.make_async_remote_copy(
                src_ref=ag_stage_ref,
                dst_ref=ag_ref.at[pl.ds(base, OWN), :],
                send_sem=ag_send_sems.at[r - 1],
                recv_sem=ag_recv_sem,
                device_id=(lax.rem(my + r, N_DEV),),
                device_id_type=pl.DeviceIdType.MESH,
            ).start()
        for t in range(N_DEV - 1):
            pltpu.make_async_copy(
                ag_ref.at[pl.ds(base, OWN), :],
                ag_ref.at[pl.ds(base, OWN), :], ag_recv_sem).wait()

        out_ref[0] = ag_ref[...].astype(jnp.float32)

        for d in range(N_DEV):
            @pl.when(my != d)
            def _(d=d):
                pltpu.make_async_copy(
                    rs_stage_ref.at[d], rs_stage_ref.at[d],
                    rs_send_sems.at[d]).wait()
        for r in range(N_DEV - 1):
            pltpu.make_async_copy(
                ag_stage_ref, ag_stage_ref, ag_send_sems.at[r]).wait()

    return pl.pallas_call(
        body,
        out_shape=jax.ShapeDtypeStruct((1, SQ, D), jnp.float32),
        in_specs=[
            pl.BlockSpec(memory_space=pltpu.VMEM),
            pl.BlockSpec(memory_space=pltpu.VMEM),
            pl.BlockSpec(memory_space=pl.ANY),
            pl.BlockSpec(memory_space=pl.ANY),
            pl.BlockSpec(memory_space=pltpu.VMEM),
        ],
        out_specs=pl.BlockSpec(memory_space=pltpu.VMEM),
        scratch_shapes=[
            pltpu.VMEM((SQ, COLS), jnp.float32),
            pltpu.VMEM((N_DEV, OWN, COLS), jnp.bfloat16),
            pltpu.VMEM((N_DEV - 1, OWN, COLS), jnp.bfloat16),
            pltpu.VMEM((OWN, D), jnp.bfloat16),
            pltpu.VMEM((SQ, D), jnp.bfloat16),
            pltpu.VMEM((SKV, D), jnp.float32),
            pltpu.VMEM((SKV, D), jnp.float32),
            pltpu.SemaphoreType.DMA((N_DEV,)),
            pltpu.SemaphoreType.DMA,
            pltpu.SemaphoreType.DMA((N_DEV - 1,)),
            pltpu.SemaphoreType.DMA,
            pltpu.SemaphoreType.DMA((2 * HQ,)),
        ],
        compiler_params=pltpu.CompilerParams(
            collective_id=0, vmem_limit_bytes=100 * 1024 * 1024),
    )(x, Wq, K_ext, V_ext, Wo)


# device time: 34940 ns/iter; 2.2932x vs baseline; 1.0493x over previous
import jax
import jax.numpy as jnp
from jax import lax
from jax.experimental import pallas as pl
from jax.experimental.pallas import tpu as pltpu

N_DEV = 8
SQ = 256
SKV = 4096
HQ = 8
DH = 128
D = HQ * DH
SCALE = 0.08838834764831843
COLS = D + DH
OWN = SQ // N_DEV
QB = 64
NG = SKV // (4 * QB)
ABLATE_COMM = False


def kernel(x, Wq, K_ext, V_ext, Wo):
    def body(x_ref, wq_ref, k_ref, v_ref, wo_ref, out_ref,
             part_ref, rs_stage_ref, rs_recv_ref, ag_stage_ref, ag_ref,
             kflat_ref, vflat_ref, kbf_ref, vbf_ref,
             rs_send_sems, rs_recv_sem, ag_send_sems, ag_recv_sem,
             ldma_sems):
        my = lax.axis_index("i")

        ldmas = []
        for h in range(HQ):
            for t, (src, dst) in enumerate(((k_ref, kflat_ref),
                                            (v_ref, vflat_ref))):
                dma = pltpu.make_async_copy(
                    src.at[0, :, h, :],
                    dst.at[:, h * DH:(h + 1) * DH],
                    ldma_sems.at[2 * h + t])
                dma.start()
                ldmas.append(dma)

        if not ABLATE_COMM:
            barrier_sem = pltpu.get_barrier_semaphore()
            for r in range(1, N_DEV):
                pl.semaphore_signal(
                    barrier_sem, inc=1,
                    device_id=(lax.rem(my + r, N_DEV),),
                    device_id_type=pl.DeviceIdType.MESH)
            pl.semaphore_wait(barrier_sem, N_DEV - 1)

        q = jnp.dot(x_ref[0].astype(jnp.bfloat16),
                    wq_ref[...].astype(jnp.bfloat16),
                    preferred_element_type=jnp.float32) * SCALE
        q_bf = q.astype(jnp.bfloat16)

        def attn_heads(c, h0, h1):
            lo, hi = h0 * DH, h1 * DH
            kgc = jnp.concatenate(
                [kbf_ref[(4 * g + c) * QB:(4 * g + c + 1) * QB, lo:hi]
                 for g in range(NG)], axis=0)
            vgc = jnp.concatenate(
                [vbf_ref[(4 * g + c) * QB:(4 * g + c + 1) * QB, lo:hi]
                 for g in range(NG)], axis=0)
            for h in range(h0, h1):
                o1, o2 = (h - h0) * DH, (h - h0 + 1) * DH
                qc = q_bf[c * QB:(c + 1) * QB, h * DH:(h + 1) * DH]
                s = lax.dot_general(
                    qc, kgc[:, o1:o2], (((1,), (1,)), ((), ())),
                    preferred_element_type=jnp.float32)
                w = jnp.exp(s)
                part_ref[c * QB:(c + 1) * QB, D + h:D + h + 1] = jnp.sum(
                    w, axis=1, keepdims=True)
                part_ref[c * QB:(c + 1) * QB, h * DH:(h + 1) * DH] = jnp.dot(
                    w.astype(jnp.bfloat16), vgc[:, o1:o2],
                    preferred_element_type=jnp.float32)

        HALF = HQ // 2 * DH
        for dma in ldmas[:HQ]:
            dma.wait()
        kbf_ref[:, 0:HALF] = kflat_ref[:, 0:HALF].astype(jnp.bfloat16)
        vbf_ref[:, 0:HALF] = vflat_ref[:, 0:HALF].astype(jnp.bfloat16)
        for c in range(4):
            attn_heads(c, 0, HQ // 2)
        for dma in ldmas[HQ:]:
            dma.wait()
        kbf_ref[:, HALF:D] = kflat_ref[:, HALF:D].astype(jnp.bfloat16)
        vbf_ref[:, HALF:D] = vflat_ref[:, HALF:D].astype(jnp.bfloat16)
        for c in range(4):
            attn_heads(c, HQ // 2, HQ)
            part_ref[c * QB:(c + 1) * QB, D + HQ:COLS] = jnp.zeros(
                (QB, COLS - D - HQ), jnp.float32)
            for d in (2 * c, 2 * c + 1):
                rs_stage_ref[d] = part_ref[
                    d * OWN:(d + 1) * OWN, :].astype(jnp.bfloat16)

                if not ABLATE_COMM:
                    @pl.when(my != d)
                    def _(d=d):
                        slot = lax.rem(jnp.int32(d + N_DEV - 1) - my, N_DEV)
                        pltpu.make_async_remote_copy(
                            src_ref=rs_stage_ref.at[d],
                            dst_ref=rs_recv_ref.at[slot],
                            send_sem=rs_send_sems.at[d],
                            recv_sem=rs_recv_sem,
                            device_id=(d,),
                            device_id_type=pl.DeviceIdType.MESH,
                        ).start()

        base = pl.multiple_of(my * OWN, 32)
        for t in range(N_DEV - 1):
            if not ABLATE_COMM:
                pltpu.make_async_copy(
                    rs_recv_ref.at[t], rs_recv_ref.at[t], rs_recv_sem).wait()
        mine = part_ref[pl.ds(base, OWN), :]
        for t in range(N_DEV - 1):
            mine = mine + rs_recv_ref[t].astype(jnp.float32)

        ctx_cols = []
        for h in range(HQ):
            ctx_cols.append(mine[:, h * DH:(h + 1) * DH]
                            / mine[:, D + h:D + h + 1])
        ctx = jnp.concatenate(ctx_cols, axis=1)
        own = jnp.dot(ctx, wo_ref[...],
                      preferred_element_type=jnp.float32)
        ag_stage_ref[...] = own.astype(jnp.bfloat16)
        ag_ref[pl.ds(base, OWN), :] = ag_stage_ref[...]

        if not ABLATE_COMM:
            for r in range(1, N_DEV):
                pltpu.make_async_remote_copy(
                    src_ref=ag_stage_ref,
                    dst_ref=ag_ref.at[pl.ds(base, OWN), :],
                    send_sem=ag_send_sems.at[r - 1],
                    recv_sem=ag_recv_sem,
                    device_id=(lax.rem(my + r, N_DEV),),
                    device_id_type=pl.DeviceIdType.MESH,
                ).start()
            for t in range(N_DEV - 1):
                pltpu.make_async_copy(
                    ag_ref.at[pl.ds(base, OWN), :],
                    ag_ref.at[pl.ds(base, OWN), :], ag_recv_sem).wait()

        out_ref[0] = ag_ref[...].astype(jnp.float32)

        if not ABLATE_COMM:
            for d in range(N_DEV):
                @pl.when(my != d)
                def _(d=d):
                    pltpu.make_async_copy(
                        rs_stage_ref.at[d], rs_stage_ref.at[d],
                        rs_send_sems.at[d]).wait()
            for r in range(N_DEV - 1):
                pltpu.make_async_copy(
                    ag_stage_ref, ag_stage_ref, ag_send_sems.at[r]).wait()

    return pl.pallas_call(
        body,
        out_shape=jax.ShapeDtypeStruct((1, SQ, D), jnp.float32),
        in_specs=[
            pl.BlockSpec(memory_space=pltpu.VMEM),
            pl.BlockSpec(memory_space=pltpu.VMEM),
            pl.BlockSpec(memory_space=pl.ANY),
            pl.BlockSpec(memory_space=pl.ANY),
            pl.BlockSpec(memory_space=pltpu.VMEM),
        ],
        out_specs=pl.BlockSpec(memory_space=pltpu.VMEM),
        scratch_shapes=[
            pltpu.VMEM((SQ, COLS), jnp.float32),
            pltpu.VMEM((N_DEV, OWN, COLS), jnp.bfloat16),
            pltpu.VMEM((N_DEV - 1, OWN, COLS), jnp.bfloat16),
            pltpu.VMEM((OWN, D), jnp.bfloat16),
            pltpu.VMEM((SQ, D), jnp.bfloat16),
            pltpu.VMEM((SKV, D), jnp.float32),
            pltpu.VMEM((SKV, D), jnp.float32),
            pltpu.VMEM((SKV, D), jnp.bfloat16),
            pltpu.VMEM((SKV, D), jnp.bfloat16),
            pltpu.SemaphoreType.DMA((N_DEV,)),
            pltpu.SemaphoreType.DMA,
            pltpu.SemaphoreType.DMA((N_DEV - 1,)),
            pltpu.SemaphoreType.DMA,
            pltpu.SemaphoreType.DMA((2 * HQ,)),
        ],
        compiler_params=pltpu.CompilerParams(
            collective_id=None if ABLATE_COMM else 0,
            vmem_limit_bytes=100 * 1024 * 1024),
    )(x, Wq, K_ext, V_ext, Wo)


# device time: 33884 ns/iter; 2.3647x vs baseline; 1.0312x over previous
import jax
import jax.numpy as jnp
from jax import lax
from jax.experimental import pallas as pl
from jax.experimental.pallas import tpu as pltpu

N_DEV = 8
SQ = 256
SKV = 4096
HQ = 8
DH = 128
D = HQ * DH
SCALE = 0.08838834764831843
COLS = D + DH
OWN = SQ // N_DEV
QB = 64
NG = SKV // (4 * QB)
ABLATE_COMM = False


def kernel(x, Wq, K_ext, V_ext, Wo):
    def body(x_ref, wq_ref, k_ref, v_ref, wo_ref, out_ref,
             part_ref, rs_stage_ref, rs_recv_ref, ag_stage_ref, ag_ref,
             kflat_ref, vflat_ref,
             rs_send_sems, rs_recv_sem, ag_send_sems, ag_recv_sem,
             ldma_sems):
        my = lax.axis_index("i")

        ldmas = []
        for h in range(HQ):
            for t, (src, dst) in enumerate(((k_ref, kflat_ref),
                                            (v_ref, vflat_ref))):
                dma = pltpu.make_async_copy(
                    src.at[0, :, h, :],
                    dst.at[:, h * DH:(h + 1) * DH],
                    ldma_sems.at[2 * h + t])
                dma.start()
                ldmas.append(dma)

        if not ABLATE_COMM:
            barrier_sem = pltpu.get_barrier_semaphore()
            for r in range(1, N_DEV):
                pl.semaphore_signal(
                    barrier_sem, inc=1,
                    device_id=(lax.rem(my + r, N_DEV),),
                    device_id_type=pl.DeviceIdType.MESH)
            pl.semaphore_wait(barrier_sem, N_DEV - 1)

        q = jnp.dot(x_ref[0], wq_ref[...],
                    preferred_element_type=jnp.float32) * SCALE

        def attn_heads(c, h0, h1):
            lo, hi = h0 * DH, h1 * DH
            kgc = jnp.concatenate(
                [kflat_ref[(4 * g + c) * QB:(4 * g + c + 1) * QB, lo:hi]
                 for g in range(NG)], axis=0)
            vgc = jnp.concatenate(
                [vflat_ref[(4 * g + c) * QB:(4 * g + c + 1) * QB, lo:hi]
                 for g in range(NG)], axis=0)
            l_cols = []
            for h in range(h0, h1):
                o1, o2 = (h - h0) * DH, (h - h0 + 1) * DH
                qc = q[c * QB:(c + 1) * QB, h * DH:(h + 1) * DH]
                s = lax.dot_general(
                    qc, kgc[:, o1:o2], (((1,), (1,)), ((), ())),
                    preferred_element_type=jnp.float32)
                w = jnp.exp(s)
                l_cols.append(jnp.sum(w, axis=1, keepdims=True))
                part_ref[c * QB:(c + 1) * QB, h * DH:(h + 1) * DH] = jnp.dot(
                    w, vgc[:, o1:o2],
                    preferred_element_type=jnp.float32)
            part_ref[c * QB:(c + 1) * QB, D + h0:D + h1] = jnp.concatenate(
                l_cols, axis=1)

        for dma in ldmas[:HQ]:
            dma.wait()
        for c in range(4):
            attn_heads(c, 0, HQ // 2)
        for dma in ldmas[HQ:]:
            dma.wait()
        for c in range(4):
            attn_heads(c, HQ // 2, HQ)
            for d in (2 * c, 2 * c + 1):
                rs_stage_ref[d] = part_ref[
                    d * OWN:(d + 1) * OWN, :].astype(jnp.bfloat16)

                if not ABLATE_COMM:
                    @pl.when(my != d)
                    def _(d=d):
                        slot = lax.rem(jnp.int32(d + N_DEV - 1) - my, N_DEV)
                        pltpu.make_async_remote_copy(
                            src_ref=rs_stage_ref.at[d],
                            dst_ref=rs_recv_ref.at[slot],
                            send_sem=rs_send_sems.at[d],
                            recv_sem=rs_recv_sem,
                            device_id=(d,),
                            device_id_type=pl.DeviceIdType.MESH,
                        ).start()

        base = pl.multiple_of(my * OWN, 32)
        for t in range(N_DEV - 1):
            if not ABLATE_COMM:
                pltpu.make_async_copy(
                    rs_recv_ref.at[t], rs_recv_ref.at[t], rs_recv_sem).wait()
        mine = part_ref[pl.ds(base, OWN), :]
        for t in range(N_DEV - 1):
            mine = mine + rs_recv_ref[t].astype(jnp.float32)

        ctx_cols = []
        for h in range(HQ):
            ctx_cols.append(mine[:, h * DH:(h + 1) * DH]
                            / mine[:, D + h:D + h + 1])
        ctx = jnp.concatenate(ctx_cols, axis=1)
        own = jnp.dot(ctx, wo_ref[...],
                      preferred_element_type=jnp.float32)
        ag_stage_ref[...] = own.astype(jnp.bfloat16)
        ag_ref[pl.ds(base, OWN), :] = ag_stage_ref[...]

        if not ABLATE_COMM:
            for r in range(1, N_DEV):
                pltpu.make_async_remote_copy(
                    src_ref=ag_stage_ref,
                    dst_ref=ag_ref.at[pl.ds(base, OWN), :],
                    send_sem=ag_send_sems.at[r - 1],
                    recv_sem=ag_recv_sem,
                    device_id=(lax.rem(my + r, N_DEV),),
                    device_id_type=pl.DeviceIdType.MESH,
                ).start()
            for t in range(N_DEV - 1):
                pltpu.make_async_copy(
                    ag_ref.at[pl.ds(base, OWN), :],
                    ag_ref.at[pl.ds(base, OWN), :], ag_recv_sem).wait()

        out_ref[0] = ag_ref[...].astype(jnp.float32)

        if not ABLATE_COMM:
            for d in range(N_DEV):
                @pl.when(my != d)
                def _(d=d):
                    pltpu.make_async_copy(
                        rs_stage_ref.at[d], rs_stage_ref.at[d],
                        rs_send_sems.at[d]).wait()
            for r in range(N_DEV - 1):
                pltpu.make_async_copy(
                    ag_stage_ref, ag_stage_ref, ag_send_sems.at[r]).wait()

    return pl.pallas_call(
        body,
        out_shape=jax.ShapeDtypeStruct((1, SQ, D), jnp.float32),
        in_specs=[
            pl.BlockSpec(memory_space=pltpu.VMEM),
            pl.BlockSpec(memory_space=pltpu.VMEM),
            pl.BlockSpec(memory_space=pl.ANY),
            pl.BlockSpec(memory_space=pl.ANY),
            pl.BlockSpec(memory_space=pltpu.VMEM),
        ],
        out_specs=pl.BlockSpec(memory_space=pltpu.VMEM),
        scratch_shapes=[
            pltpu.VMEM((SQ, COLS), jnp.float32),
            pltpu.VMEM((N_DEV, OWN, COLS), jnp.bfloat16),
            pltpu.VMEM((N_DEV - 1, OWN, COLS), jnp.bfloat16),
            pltpu.VMEM((OWN, D), jnp.bfloat16),
            pltpu.VMEM((SQ, D), jnp.bfloat16),
            pltpu.VMEM((SKV, D), jnp.float32),
            pltpu.VMEM((SKV, D), jnp.float32),
            pltpu.SemaphoreType.DMA((N_DEV,)),
            pltpu.SemaphoreType.DMA,
            pltpu.SemaphoreType.DMA((N_DEV - 1,)),
            pltpu.SemaphoreType.DMA,
            pltpu.SemaphoreType.DMA((2 * HQ,)),
        ],
        compiler_params=pltpu.CompilerParams(
            collective_id=None if ABLATE_COMM else 0,
            vmem_limit_bytes=100 * 1024 * 1024),
    )(x, Wq, K_ext, V_ext, Wo)
